# Initial kernel scaffold; baseline (speedup 1.0000x reference)
import jax
import jax.numpy as jnp
from jax.experimental import pallas as pl


def kernel(x, edge_index, W1, b1, g1, be1, W2, b2, g2, be2, Wfc, bfc):
    raise NotImplementedError("write your pallas kernel here")


# speedup vs baseline: 30.2343x; 30.2343x over previous
import functools

import jax
import jax.numpy as jnp
from jax import lax
from jax.experimental import pallas as pl
from jax.experimental.pallas import tpu as pltpu
from jax.experimental.pallas import tpu_sc as plsc

N = 10000
HID = 16
OUT_DIM = 64
EPS = 1e-5

NC, NS, LANES = 2, 16, 16
NW = NC * NS
CHUNK = 128
RPS = 632
NPAD = NS * RPS


def _sc_degree(dst_idx):
    K = dst_idx.shape[1]
    mesh = plsc.VectorSubcoreMesh(core_axis_name="c", subcore_axis_name="s",
                                  num_cores=NC, num_subcores=NS)

    @functools.partial(
        pl.kernel, mesh=mesh,
        out_type=jax.ShapeDtypeStruct((NC, NPAD, LANES), jnp.float32),
        scratch_types=[
            pltpu.VMEM((K, CHUNK), jnp.int32),
            pltpu.VMEM((CHUNK, LANES), jnp.float32),
            pltpu.VMEM((RPS, LANES), jnp.float32),
            pltpu.VMEM_SHARED((NPAD, LANES), jnp.float32),
        ],
        compiler_params=pltpu.CompilerParams(use_tc_tiling_on_sc=False))
    def k(dst_hbm, out_hbm, dstv, onesb, zbuf, acc):
        c = lax.axis_index("c")
        s = lax.axis_index("s")
        wid = c * NS + s

        def fill_zero(i, carry):
            zbuf[i, :] = jnp.zeros((LANES,), jnp.float32)
            return carry
        lax.fori_loop(0, RPS, fill_zero, None)

        def fill_one(i, carry):
            onesb[i, :] = jnp.ones((LANES,), jnp.float32)
            return carry
        lax.fori_loop(0, CHUNK, fill_one, None)

        pltpu.sync_copy(zbuf, acc.at[pl.ds(s * RPS, RPS)])
        pltpu.sync_copy(dst_hbm.at[wid], dstv)
        plsc.subcore_barrier()

        def body(j, carry):
            pltpu.sync_copy(onesb, acc.at[dstv.at[j]], add=True)
            return carry
        lax.fori_loop(0, K, body, None)

        plsc.subcore_barrier()
        pltpu.sync_copy(acc.at[pl.ds(s * RPS, RPS)],
                        out_hbm.at[c, pl.ds(s * RPS, RPS)])

    return k(dst_idx)


def _sc_scatter_rows(src_idx, dst_idx, table):
    K = src_idx.shape[1]
    mesh = plsc.VectorSubcoreMesh(core_axis_name="c", subcore_axis_name="s",
                                  num_cores=NC, num_subcores=NS)

    @functools.partial(
        pl.kernel, mesh=mesh,
        out_type=jax.ShapeDtypeStruct((NC, NPAD, LANES), jnp.float32),
        scratch_types=[
            pltpu.VMEM((K, CHUNK), jnp.int32),
            pltpu.VMEM((K, CHUNK), jnp.int32),
            pltpu.VMEM((CHUNK, LANES), jnp.float32),
            pltpu.VMEM((RPS, LANES), jnp.float32),
            pltpu.VMEM_SHARED((NPAD, LANES), jnp.float32),
            pltpu.SemaphoreType.DMA,
        ],
        compiler_params=pltpu.CompilerParams(use_tc_tiling_on_sc=False))
    def k(src_hbm, dst_hbm, tab_hbm, out_hbm, srcv, dstv, rows, zbuf, acc, sem):
        c = lax.axis_index("c")
        s = lax.axis_index("s")
        wid = c * NS + s

        def fill_zero(i, carry):
            zbuf[i, :] = jnp.zeros((LANES,), jnp.float32)
            return carry
        lax.fori_loop(0, RPS, fill_zero, None)

        pltpu.sync_copy(zbuf, acc.at[pl.ds(s * RPS, RPS)])
        pltpu.sync_copy(src_hbm.at[wid], srcv)
        pltpu.sync_copy(dst_hbm.at[wid], dstv)
        plsc.subcore_barrier()

        def body(j, carry):
            pltpu.async_copy(tab_hbm.at[srcv.at[j]], rows, sem).wait()
            pltpu.sync_copy(rows, acc.at[dstv.at[j]], add=True)
            return carry
        lax.fori_loop(0, K, body, None)

        plsc.subcore_barrier()
        pltpu.sync_copy(acc.at[pl.ds(s * RPS, RPS)],
                        out_hbm.at[c, pl.ds(s * RPS, RPS)])

    return k(src_idx, dst_idx, table)


def _tc_prep(xp, W1, dacc):
    def body(x_ref, w_ref, da_ref, db_ref, dinv_ref, dhv_ref):
        deg = da_ref[...] + db_ref[...] + 1.0
        rows = lax.broadcasted_iota(jnp.int32, (NPAD, HID), 0)
        dinv = jnp.where(rows < N, lax.rsqrt(deg), 0.0)
        h = jnp.dot(x_ref[...], w_ref[...], preferred_element_type=jnp.float32)
        dinv_ref[...] = dinv
        dhv_ref[...] = dinv * h

    return pl.pallas_call(
        body,
        out_shape=(jax.ShapeDtypeStruct((NPAD, HID), jnp.float32),
                   jax.ShapeDtypeStruct((NPAD, HID), jnp.float32)),
    )(xp, W1, dacc[0], dacc[1])


def _tc_mid(acc, dhv, dinv, b, g, be, W):
    def body(a_ref, b2_ref, dhv_ref, dinv_ref, bias_ref, g_ref, be_ref, w_ref,
             out_ref):
        rows = lax.broadcasted_iota(jnp.int32, (NPAD, HID), 0)
        valid = rows < N
        dinv = dinv_ref[...]
        s = dinv * (a_ref[...] + b2_ref[...] + dhv_ref[...]) + bias_ref[...]
        sv = jnp.where(valid, s, 0.0)
        mean = jnp.sum(sv, axis=0, keepdims=True) * (1.0 / N)
        d = s - mean
        var = jnp.sum(jnp.where(valid, d * d, 0.0), axis=0, keepdims=True) * (1.0 / N)
        bn = d * lax.rsqrt(var + EPS) * g_ref[...] + be_ref[...]
        h = jnp.where(valid, jnp.maximum(bn, 0.0), 0.0)
        out_ref[...] = dinv * jnp.dot(h, w_ref[...],
                                      preferred_element_type=jnp.float32)

    return pl.pallas_call(
        body,
        out_shape=jax.ShapeDtypeStruct((NPAD, HID), jnp.float32),
    )(acc[0], acc[1], dhv, dinv, b, g, be, W)


def _tc_final(acc, dhv, dinv, b, g, be, Wfc, bfc):
    def body(a_ref, b2_ref, dhv_ref, dinv_ref, bias_ref, g_ref, be_ref, w_ref,
             bf_ref, out_ref):
        rows = lax.broadcasted_iota(jnp.int32, (NPAD, HID), 0)
        valid = rows < N
        s = dinv_ref[...] * (a_ref[...] + b2_ref[...] + dhv_ref[...]) + bias_ref[...]
        sv = jnp.where(valid, s, 0.0)
        mean = jnp.sum(sv, axis=0, keepdims=True) * (1.0 / N)
        d = s - mean
        var = jnp.sum(jnp.where(valid, d * d, 0.0), axis=0, keepdims=True) * (1.0 / N)
        bn = d * lax.rsqrt(var + EPS) * g_ref[...] + be_ref[...]
        h = jnp.where(valid, jnp.maximum(bn, 0.0), 0.0)
        out_ref[...] = jnp.dot(h, w_ref[...],
                               preferred_element_type=jnp.float32) + bf_ref[...]

    return pl.pallas_call(
        body,
        out_shape=jax.ShapeDtypeStruct((NPAD, OUT_DIM), jnp.float32),
    )(acc[0], acc[1], dhv, dinv, b, g, be, Wfc, bfc)


def kernel(x, edge_index, W1, b1, g1, be1, W2, b2, g2, be2, Wfc, bfc):
    E = edge_index.shape[1]
    K = -(-E // (NW * CHUNK))
    EP = NW * CHUNK * K

    ei = edge_index.astype(jnp.int32)
    pad = jnp.full((EP - E,), NPAD - 1, jnp.int32)
    src = jnp.concatenate([ei[0], pad]).reshape(NW, K, CHUNK)
    dst = jnp.concatenate([ei[1], pad]).reshape(NW, K, CHUNK)
    xp = jnp.pad(x, ((0, NPAD - N), (0, 0)))

    dacc = _sc_degree(dst)
    dinv, ht1 = _tc_prep(xp, W1, dacc)
    acc1 = _sc_scatter_rows(src, dst, ht1)
    ht2 = _tc_mid(acc1, ht1, dinv, b1.reshape(1, -1), g1.reshape(1, -1),
                  be1.reshape(1, -1), W2)
    acc2 = _sc_scatter_rows(src, dst, ht2)
    out = _tc_final(acc2, ht2, dinv, b2.reshape(1, -1), g2.reshape(1, -1),
                    be2.reshape(1, -1), Wfc, bfc.reshape(1, -1))
    return out[:N]
